# Initial kernel scaffold; baseline (speedup 1.0000x reference)
import functools

import jax
import jax.numpy as jnp
from jax import lax
from jax.experimental import pallas as pl
from jax.experimental.pallas import tpu as pltpu
from jax.experimental.pallas import tpu_sc as plsc

N = 10000
NP = 10240
E = 320000
EROWS = 2528
EP = EROWS * 128
ROWS_PER = EROWS // 32
G = 256
D1 = 32
D2 = 64

_MESH = plsc.VectorSubcoreMesh(core_axis_name="c", subcore_axis_name="s")


@functools.partial(
    pl.kernel,
    out_type=jax.ShapeDtypeStruct((32, NP), jnp.float32),
    mesh=_MESH,
    scratch_types=[
        pltpu.VMEM((ROWS_PER, 128), jnp.int32),
        pltpu.VMEM((NP,), jnp.float32),
    ],
)
def _deg_kernel(dst_hbm, degp_hbm, dst_v, hist_v):
    cid = lax.axis_index("c")
    sid = lax.axis_index("s")
    w = cid * 16 + sid

    def zero_body(i, carry):
        hist_v[pl.ds(i * 16, 16)] = jnp.zeros((16,), jnp.float32)
        return carry

    lax.fori_loop(0, NP // 16, zero_body, 0)

    pltpu.sync_copy(dst_hbm.at[pl.ds(w * ROWS_PER, ROWS_PER)], dst_v)

    ones = jnp.ones((16,), jnp.float32)

    def row_body(k, carry):
        for j in range(8):
            idx = dst_v[k, pl.ds(j * 16, 16)]
            plsc.addupdate_scatter(hist_v, [idx], ones)
        return carry

    lax.fori_loop(0, ROWS_PER, row_body, 0)
    pltpu.sync_copy(hist_v, degp_hbm.at[w])


def _make_scatter(D):
    @functools.partial(
        pl.kernel,
        out_type=jax.ShapeDtypeStruct((2, NP, D), jnp.float32),
        mesh=_MESH,
        scratch_types=[
            pltpu.VMEM((ROWS_PER, 128), jnp.int32),
            pltpu.VMEM((ROWS_PER, 128), jnp.int32),
            pltpu.VMEM((128, D), jnp.float32),
            pltpu.VMEM_SHARED((NP, D), jnp.float32),
        ],
    )
    def scat(src_hbm, dst_hbm, g_hbm, accp_hbm, src_v, dst_v, rows_v, acc_sh):
        cid = lax.axis_index("c")
        sid = lax.axis_index("s")
        w = cid * 16 + sid

        def zrow(i, carry):
            for j in range(D // 16):
                rows_v[i, pl.ds(j * 16, 16)] = jnp.zeros((16,), jnp.float32)
            return carry

        lax.fori_loop(0, 128, zrow, 0)
        for t in range(5):
            pltpu.sync_copy(rows_v, acc_sh.at[pl.ds(sid * 640 + t * 128, 128), :])
        plsc.subcore_barrier()

        pltpu.sync_copy(src_hbm.at[pl.ds(w * ROWS_PER, ROWS_PER)], src_v)
        pltpu.sync_copy(dst_hbm.at[pl.ds(w * ROWS_PER, ROWS_PER)], dst_v)

        def edge_body(k, carry):
            pltpu.sync_copy(g_hbm.at[src_v.at[k]], rows_v)
            pltpu.sync_copy(rows_v, acc_sh.at[dst_v.at[k]], add=True)
            return carry

        lax.fori_loop(0, ROWS_PER, edge_body, 0)
        plsc.subcore_barrier()

        for t in range(5):
            r0 = sid * 640 + t * 128
            pltpu.sync_copy(acc_sh.at[pl.ds(r0, 128), :], rows_v)
            pltpu.sync_copy(rows_v, accp_hbm.at[cid, pl.ds(r0, 128), :])

    return scat


_scatter32 = _make_scatter(D1)
_scatter64 = _make_scatter(D2)


def _tc1_body(x_ref, w1_ref, degp_ref, g1_ref, dinv_ref):
    deg = jnp.sum(degp_ref[...], axis=0)[:N] + 1.0
    dinv = lax.rsqrt(deg).reshape(N, 1)
    h = jnp.dot(x_ref[...], w1_ref[...], preferred_element_type=jnp.float32)
    g1_ref[...] = h * dinv
    dinv_ref[...] = dinv


def _tc1(x, W1, degp):
    return pl.pallas_call(
        _tc1_body,
        out_shape=[
            jax.ShapeDtypeStruct((N, D1), jnp.float32),
            jax.ShapeDtypeStruct((N, 1), jnp.float32),
        ],
    )(x, W1, degp)


def _tc2_body(g1_ref, accp_ref, dinv_ref, b1_ref, w2_ref, g2_ref):
    acc = accp_ref[0, :N, :] + accp_ref[1, :N, :] + g1_ref[...]
    dinv = dinv_ref[...]
    o = jnp.maximum(acc * dinv + b1_ref[...][None, :], 0.0)
    h2 = jnp.dot(o, w2_ref[...], preferred_element_type=jnp.float32)
    g2_ref[...] = h2 * dinv


def _tc2(g1, accp1, dinv, b1, W2):
    return pl.pallas_call(
        _tc2_body,
        out_shape=jax.ShapeDtypeStruct((N, D2), jnp.float32),
    )(g1, accp1, dinv, b1, W2)


def _tc3_body(g2_ref, accp_ref, dinv_ref, b2_ref, bi_ref, out_ref):
    acc = accp_ref[0, :N, :] + accp_ref[1, :N, :] + g2_ref[...]
    pre = acc * dinv_ref[...] + b2_ref[...][None, :]
    sp = jnp.maximum(pre, 0.0) + jnp.log1p(jnp.exp(-jnp.abs(pre)))
    m = pre * jnp.tanh(sp)
    gid = lax.broadcasted_iota(jnp.int32, (1, G), 1)
    onehot = (bi_ref[...] == gid).astype(jnp.float32)
    sums = lax.dot_general(
        onehot, m, dimension_numbers=(((0,), (0,)), ((), ())),
        preferred_element_type=jnp.float32,
    )
    cnt = jnp.sum(onehot, axis=0)
    out_ref[...] = sums / jnp.maximum(cnt, 1.0)[:, None]


def _tc3(g2, accp2, dinv, b2, bi2d):
    return pl.pallas_call(
        _tc3_body,
        out_shape=jax.ShapeDtypeStruct((G, D2), jnp.float32),
    )(g2, accp2, dinv, b2, bi2d)


def kernel(x, edge_index, batch_index, W1, b1, W2, b2):
    src = edge_index[0]
    dst = edge_index[1]
    pad = EP - E
    srcp = jnp.concatenate([src, jnp.zeros((pad,), jnp.int32)]).reshape(EROWS, 128)
    dstp = jnp.concatenate(
        [dst, jnp.full((pad,), N, jnp.int32)]).reshape(EROWS, 128)

    degp = _deg_kernel(dstp)
    g1, dinv = _tc1(x, W1, degp)
    accp1 = _scatter32(srcp, dstp, g1)
    g2 = _tc2(g1, accp1, dinv, b1, W2)
    accp2 = _scatter64(srcp, dstp, g2)
    return _tc3(g2, accp2, dinv, b2, batch_index.reshape(N, 1))


# speedup vs baseline: 18.6585x; 18.6585x over previous
import functools

import jax
import jax.numpy as jnp
from jax import lax
from jax.experimental import pallas as pl
from jax.experimental.pallas import tpu as pltpu
from jax.experimental.pallas import tpu_sc as plsc

N = 10000
NP = 10240
E = 320000
EROWS = 2560
EP = EROWS * 128
ROWS_PER = EROWS // 32
G = 256
D1 = 32
D2 = 64

DW = 16


@functools.cache
def _get_deg_kernel():
    mesh = plsc.VectorSubcoreMesh(core_axis_name="c", subcore_axis_name="s")
    return functools.partial(
        pl.kernel,
        out_type=jax.ShapeDtypeStruct((2, NP, DW), jnp.float32),
        mesh=mesh,
        scratch_types=[
            pltpu.VMEM((ROWS_PER, 128), jnp.int32),
            pltpu.VMEM((128, DW), jnp.float32),
            pltpu.VMEM_SHARED((NP, DW), jnp.float32),
        ],
    )(_deg_body)


def _deg_body(dst_hbm, degp_hbm, dst_v, ones_v, deg_sh):
    cid = lax.axis_index("c")
    sid = lax.axis_index("s")
    w = cid * 16 + sid

    def fill(val):
        def body(i, carry):
            ones_v[i, pl.ds(0, 16)] = jnp.full((16,), val, jnp.float32)
            return carry
        return body

    lax.fori_loop(0, 128, fill(0.0), 0)
    for t in range(5):
        pltpu.sync_copy(ones_v, deg_sh.at[pl.ds(sid * 640 + t * 128, 128), :])
    plsc.subcore_barrier()

    lax.fori_loop(0, 128, fill(1.0), 0)
    pltpu.sync_copy(dst_hbm.at[pl.ds(w * ROWS_PER, ROWS_PER)], dst_v)

    def row_body(k, carry):
        pltpu.sync_copy(ones_v, deg_sh.at[dst_v.at[k]], add=True)
        return carry

    lax.fori_loop(0, ROWS_PER, row_body, 0)
    plsc.subcore_barrier()

    for t in range(5):
        r0 = sid * 640 + t * 128
        pltpu.sync_copy(deg_sh.at[pl.ds(r0, 128), :], ones_v)
        pltpu.sync_copy(ones_v, degp_hbm.at[cid, pl.ds(r0, 128), :])


@functools.cache
def _make_scatter(D):
    mesh = plsc.VectorSubcoreMesh(core_axis_name="c", subcore_axis_name="s")

    @functools.partial(
        pl.kernel,
        out_type=jax.ShapeDtypeStruct((2, NP, D), jnp.float32),
        mesh=mesh,
        scratch_types=[
            pltpu.VMEM((ROWS_PER, 128), jnp.int32),
            pltpu.VMEM((ROWS_PER, 128), jnp.int32),
            pltpu.VMEM((128, D), jnp.float32),
            pltpu.VMEM_SHARED((NP, D), jnp.float32),
        ],
        compiler_params=pltpu.CompilerParams(use_tc_tiling_on_sc=False),
    )
    def scat(src_hbm, dst_hbm, g_hbm, accp_hbm, src_v, dst_v, rows_v, acc_sh):
        cid = lax.axis_index("c")
        sid = lax.axis_index("s")
        w = cid * 16 + sid

        def zrow(i, carry):
            for j in range(D // 16):
                rows_v[i, pl.ds(j * 16, 16)] = jnp.zeros((16,), jnp.float32)
            return carry

        lax.fori_loop(0, 128, zrow, 0)
        for t in range(5):
            pltpu.sync_copy(rows_v, acc_sh.at[pl.ds(sid * 640 + t * 128, 128), :])
        plsc.subcore_barrier()

        pltpu.sync_copy(src_hbm.at[pl.ds(w * ROWS_PER, ROWS_PER)], src_v)
        pltpu.sync_copy(dst_hbm.at[pl.ds(w * ROWS_PER, ROWS_PER)], dst_v)

        def edge_body(k, carry):
            pltpu.sync_copy(g_hbm.at[src_v.at[k]], rows_v)
            pltpu.sync_copy(rows_v, acc_sh.at[dst_v.at[k]], add=True)
            return carry

        lax.fori_loop(0, ROWS_PER, edge_body, 0)
        plsc.subcore_barrier()

        for t in range(5):
            r0 = sid * 640 + t * 128
            pltpu.sync_copy(acc_sh.at[pl.ds(r0, 128), :], rows_v)
            pltpu.sync_copy(rows_v, accp_hbm.at[cid, pl.ds(r0, 128), :])

    return scat


def _tc1_body(x_ref, w1_ref, degp_ref, g1_ref, dinv_ref):
    deg = degp_ref[0, :N, 0] + degp_ref[1, :N, 0] + 1.0
    dinv = lax.rsqrt(deg).reshape(N, 1)
    h = jnp.dot(x_ref[...], w1_ref[...], preferred_element_type=jnp.float32)
    g1_ref[...] = h * dinv
    dinv_ref[...] = dinv


def _tc1(x, W1, degp):
    return pl.pallas_call(
        _tc1_body,
        out_shape=[
            jax.ShapeDtypeStruct((N, D1), jnp.float32),
            jax.ShapeDtypeStruct((N, 1), jnp.float32),
        ],
    )(x, W1, degp)


def _tc2_body(g1_ref, accp_ref, dinv_ref, b1_ref, w2_ref, g2_ref):
    acc = accp_ref[0, :N, :] + accp_ref[1, :N, :] + g1_ref[...]
    dinv = dinv_ref[...]
    o = jnp.maximum(acc * dinv + b1_ref[...][None, :], 0.0)
    h2 = jnp.dot(o, w2_ref[...], preferred_element_type=jnp.float32)
    g2_ref[...] = h2 * dinv


def _tc2(g1, accp1, dinv, b1, W2):
    return pl.pallas_call(
        _tc2_body,
        out_shape=jax.ShapeDtypeStruct((N, D2), jnp.float32),
    )(g1, accp1, dinv, b1, W2)


def _tc3_body(g2_ref, accp_ref, dinv_ref, b2_ref, bi_ref, out_ref):
    acc = accp_ref[0, :N, :] + accp_ref[1, :N, :] + g2_ref[...]
    pre = acc * dinv_ref[...] + b2_ref[...][None, :]
    sp = jnp.maximum(pre, 0.0) + jnp.log1p(jnp.exp(-jnp.abs(pre)))
    m = pre * jnp.tanh(sp)
    gid = lax.broadcasted_iota(jnp.int32, (1, G), 1)
    onehot = (bi_ref[...] == gid).astype(jnp.float32)
    sums = lax.dot_general(
        onehot, m, dimension_numbers=(((0,), (0,)), ((), ())),
        preferred_element_type=jnp.float32,
    )
    cnt = jnp.sum(onehot, axis=0)
    out_ref[...] = sums / jnp.maximum(cnt, 1.0)[:, None]


def _tc3(g2, accp2, dinv, b2, bi2d):
    return pl.pallas_call(
        _tc3_body,
        out_shape=jax.ShapeDtypeStruct((G, D2), jnp.float32),
    )(g2, accp2, dinv, b2, bi2d)


def kernel(x, edge_index, batch_index, W1, b1, W2, b2):
    src = edge_index[0]
    dst = edge_index[1]
    pad = EP - E
    srcp = jnp.concatenate([src, jnp.zeros((pad,), jnp.int32)]).reshape(EROWS, 128)
    dstp = jnp.concatenate(
        [dst, jnp.full((pad,), N, jnp.int32)]).reshape(EROWS, 128)

    degp = _get_deg_kernel()(dstp)
    g1, dinv = _tc1(x, W1, degp)
    accp1 = _make_scatter(D1)(srcp, dstp, g1)
    g2 = _tc2(g1, accp1, dinv, b1, W2)
    accp2 = _make_scatter(D2)(srcp, dstp, g2)
    return _tc3(g2, accp2, dinv, b2, batch_index.reshape(N, 1))


# speedup vs baseline: 20.0112x; 1.0725x over previous
import functools

import jax
import jax.numpy as jnp
from jax import lax
from jax.experimental import pallas as pl
from jax.experimental.pallas import tpu as pltpu
from jax.experimental.pallas import tpu_sc as plsc

N = 10000
NP = 10240
E = 320000
EROWS = 2560
EP = EROWS * 128
ROWS_PER = EROWS // 32
G = 256
D1 = 32
D2 = 64

DW = 16


@functools.cache
def _get_deg_kernel():
    mesh = plsc.VectorSubcoreMesh(core_axis_name="c", subcore_axis_name="s")
    return functools.partial(
        pl.kernel,
        out_type=jax.ShapeDtypeStruct((2, NP, DW), jnp.float32),
        mesh=mesh,
        scratch_types=[
            pltpu.VMEM((ROWS_PER, 128), jnp.int32),
            pltpu.VMEM((640, DW), jnp.float32),
            pltpu.VMEM_SHARED((NP, DW), jnp.float32),
        ],
        compiler_params=pltpu.CompilerParams(use_tc_tiling_on_sc=False),
    )(_deg_body)


def _deg_body(dst_hbm, degp_hbm, dst_v, ones_v, deg_sh):
    cid = lax.axis_index("c")
    sid = lax.axis_index("s")
    w = cid * 16 + sid

    def fill(val, n):
        def body(i, carry):
            ones_v[i, pl.ds(0, 16)] = jnp.full((16,), val, jnp.float32)
            return carry
        return body

    lax.fori_loop(0, 640, fill(0.0, 640), 0)
    pltpu.sync_copy(ones_v, deg_sh.at[pl.ds(sid * 640, 640), :])
    plsc.subcore_barrier()

    lax.fori_loop(0, 128, fill(1.0, 128), 0)
    pltpu.sync_copy(dst_hbm.at[pl.ds(w * ROWS_PER, ROWS_PER)], dst_v)

    def row_body(k, carry):
        pltpu.sync_copy(ones_v.at[pl.ds(0, 128), :],
                        deg_sh.at[dst_v.at[k]], add=True)
        return carry

    lax.fori_loop(0, ROWS_PER, row_body, 0)
    plsc.subcore_barrier()

    pltpu.sync_copy(deg_sh.at[pl.ds(sid * 640, 640), :], ones_v)
    pltpu.sync_copy(ones_v, degp_hbm.at[cid, pl.ds(sid * 640, 640), :])


NB = 8


@functools.cache
def _make_scatter(D):
    mesh = plsc.VectorSubcoreMesh(core_axis_name="c", subcore_axis_name="s")
    K = ROWS_PER

    @functools.partial(
        pl.kernel,
        out_type=jax.ShapeDtypeStruct((2, NP, D), jnp.float32),
        mesh=mesh,
        scratch_types=[
            pltpu.VMEM((ROWS_PER, 128), jnp.int32),
            pltpu.VMEM((ROWS_PER, 128), jnp.int32),
            [pltpu.VMEM((128, D), jnp.float32) for _ in range(NB)],
            pltpu.VMEM_SHARED((NP, D), jnp.float32),
            [pltpu.SemaphoreType.DMA for _ in range(NB)],
        ],
        compiler_params=pltpu.CompilerParams(use_tc_tiling_on_sc=False),
    )
    def scat(src_hbm, dst_hbm, g_hbm, accp_hbm,
             src_v, dst_v, bufs, acc_sh, sems):
        cid = lax.axis_index("c")
        sid = lax.axis_index("s")
        w = cid * 16 + sid

        def zrow(i, carry):
            for j in range(D // 16):
                bufs[0][i, pl.ds(j * 16, 16)] = jnp.zeros((16,), jnp.float32)
            return carry

        lax.fori_loop(0, 128, zrow, 0)
        for t in range(5):
            pltpu.sync_copy(bufs[0], acc_sh.at[pl.ds(sid * 640 + t * 128, 128), :])
        plsc.subcore_barrier()

        pltpu.sync_copy(src_hbm.at[pl.ds(w * ROWS_PER, ROWS_PER)], src_v)
        pltpu.sync_copy(dst_hbm.at[pl.ds(w * ROWS_PER, ROWS_PER)], dst_v)

        def gather_start(k, b):
            return pltpu.async_copy(g_hbm.at[src_v.at[k]], bufs[b], sems[b])

        def gather_wait(b):
            pltpu.make_async_copy(g_hbm.at[src_v.at[0]], bufs[b], sems[b]).wait()

        for b in range(NB):
            gather_start(b, b)

        def group_body(g, carry):
            for b in range(NB):
                k = g * NB + b
                gather_wait(b)
                pltpu.sync_copy(bufs[b], acc_sh.at[dst_v.at[k]], add=True)
                gather_start(k + NB, b)
            return carry

        lax.fori_loop(0, K // NB - 1, group_body, 0)

        for b in range(NB):
            k = K - NB + b
            gather_wait(b)
            pltpu.sync_copy(bufs[b], acc_sh.at[dst_v.at[k]], add=True)
        plsc.subcore_barrier()

        for t in range(5):
            r0 = sid * 640 + t * 128
            pltpu.sync_copy(acc_sh.at[pl.ds(r0, 128), :], bufs[0])
            pltpu.sync_copy(bufs[0], accp_hbm.at[cid, pl.ds(r0, 128), :])

    return scat


def _tc1_body(x_ref, w1_ref, degp_ref, g1_ref, dinv_ref):
    deg = degp_ref[0, :N, 0] + degp_ref[1, :N, 0] + 1.0
    dinv = lax.rsqrt(deg).reshape(N, 1)
    h = jnp.dot(x_ref[...], w1_ref[...], preferred_element_type=jnp.float32)
    g1_ref[...] = h * dinv
    dinv_ref[...] = dinv


def _tc1(x, W1, degp):
    return pl.pallas_call(
        _tc1_body,
        out_shape=[
            jax.ShapeDtypeStruct((N, D1), jnp.float32),
            jax.ShapeDtypeStruct((N, 1), jnp.float32),
        ],
    )(x, W1, degp)


def _tc2_body(g1_ref, accp_ref, dinv_ref, b1_ref, w2_ref, g2_ref):
    acc = accp_ref[0, :N, :] + accp_ref[1, :N, :] + g1_ref[...]
    dinv = dinv_ref[...]
    o = jnp.maximum(acc * dinv + b1_ref[...][None, :], 0.0)
    h2 = jnp.dot(o, w2_ref[...], preferred_element_type=jnp.float32)
    g2_ref[...] = h2 * dinv


def _tc2(g1, accp1, dinv, b1, W2):
    return pl.pallas_call(
        _tc2_body,
        out_shape=jax.ShapeDtypeStruct((N, D2), jnp.float32),
    )(g1, accp1, dinv, b1, W2)


def _tc3_body(g2_ref, accp_ref, dinv_ref, b2_ref, bi_ref, out_ref):
    acc = accp_ref[0, :N, :] + accp_ref[1, :N, :] + g2_ref[...]
    pre = acc * dinv_ref[...] + b2_ref[...][None, :]
    sp = jnp.maximum(pre, 0.0) + jnp.log1p(jnp.exp(-jnp.abs(pre)))
    m = pre * jnp.tanh(sp)
    gid = lax.broadcasted_iota(jnp.int32, (1, G), 1)
    onehot = (bi_ref[...] == gid).astype(jnp.float32)
    sums = lax.dot_general(
        onehot, m, dimension_numbers=(((0,), (0,)), ((), ())),
        preferred_element_type=jnp.float32,
    )
    cnt = jnp.sum(onehot, axis=0)
    out_ref[...] = sums / jnp.maximum(cnt, 1.0)[:, None]


def _tc3(g2, accp2, dinv, b2, bi2d):
    return pl.pallas_call(
        _tc3_body,
        out_shape=jax.ShapeDtypeStruct((G, D2), jnp.float32),
    )(g2, accp2, dinv, b2, bi2d)


def kernel(x, edge_index, batch_index, W1, b1, W2, b2):
    src = edge_index[0]
    dst = edge_index[1]
    pad = EP - E
    srcp = jnp.concatenate([src, jnp.zeros((pad,), jnp.int32)]).reshape(EROWS, 128)
    dstp = jnp.concatenate(
        [dst, jnp.full((pad,), N, jnp.int32)]).reshape(EROWS, 128)

    degp = _get_deg_kernel()(dstp)
    g1, dinv = _tc1(x, W1, degp)
    accp1 = _make_scatter(D1)(srcp, dstp, g1)
    g2 = _tc2(g1, accp1, dinv, b1, W2)
    accp2 = _make_scatter(D2)(srcp, dstp, g2)
    return _tc3(g2, accp2, dinv, b2, batch_index.reshape(N, 1))
